# Initial kernel scaffold; baseline (speedup 1.0000x reference)
import jax
import jax.numpy as jnp
from jax.experimental import pallas as pl


def kernel(x, w1, b1, w2, b2):
    raise NotImplementedError("write your pallas kernel here")


# speedup vs baseline: 1.0114x; 1.0114x over previous
import jax
import jax.numpy as jnp
from jax.experimental import pallas as pl
from jax.experimental.pallas import tpu as pltpu


def _round_up(n, m):
    return ((n + m - 1) // m) * m


def _mlp_kernel(x_ref, w1_ref, b1_ref, w2_ref, b2_ref, o_ref):
    x = x_ref[...].astype(jnp.bfloat16)
    h = jnp.dot(x, w1_ref[...], preferred_element_type=jnp.float32)
    h = jax.nn.sigmoid(h + b1_ref[...])
    y = jnp.dot(h.astype(jnp.bfloat16), w2_ref[...],
                preferred_element_type=jnp.float32)
    o_ref[...] = (y + b2_ref[...]).astype(o_ref.dtype)


def kernel(x, w1, b1, w2, b2):
    B, D_in = x.shape
    H = w1.shape[1]
    D_out = w2.shape[1]

    pack = (128 // D_in) if (D_in < 128 and 128 % D_in == 0) else 1

    tile_p = 4096
    B_p = pl.cdiv(B, pack)
    if B_p <= tile_p:
        tile_p = _round_up(B_p, 8)
    B_pad = _round_up(B, pack * tile_p)
    if B_pad > B:
        x = jnp.pad(x, ((0, B_pad - B), (0, 0)))
    B_p = B_pad // pack

    x_p = x.reshape(B_p, pack * D_in)
    eye = jnp.eye(pack, dtype=w1.dtype)
    w1_p = jnp.kron(eye, w1).astype(jnp.bfloat16)
    w2_p = jnp.kron(eye, w2).astype(jnp.bfloat16)
    b1_p = jnp.tile(b1, (1, pack))
    b2_p = jnp.tile(b2, (1, pack))

    grid = (B_p // tile_p,)
    out_p = pl.pallas_call(
        _mlp_kernel,
        out_shape=jax.ShapeDtypeStruct((B_p, pack * D_out), x.dtype),
        grid_spec=pl.GridSpec(
            grid=grid,
            in_specs=[
                pl.BlockSpec((tile_p, pack * D_in), lambda i: (i, 0)),
                pl.BlockSpec((pack * D_in, pack * H), lambda i: (0, 0)),
                pl.BlockSpec((1, pack * H), lambda i: (0, 0)),
                pl.BlockSpec((pack * H, pack * D_out), lambda i: (0, 0)),
                pl.BlockSpec((1, pack * D_out), lambda i: (0, 0)),
            ],
            out_specs=pl.BlockSpec((tile_p, pack * D_out), lambda i: (i, 0)),
        ),
        compiler_params=pltpu.CompilerParams(
            dimension_semantics=("parallel",),
            vmem_limit_bytes=64 * 1024 * 1024,
        ),
    )(x_p, w1_p, b1_p, w2_p, b2_p)

    return out_p.reshape(B_pad, D_out)[:B]


# speedup vs baseline: 1.2551x; 1.2410x over previous
import jax
import jax.numpy as jnp
from jax.experimental import pallas as pl
from jax.experimental.pallas import tpu as pltpu


def _round_up(n, m):
    return ((n + m - 1) // m) * m


def _mlp_kernel(x_ref, w1_ref, b1_ref, w2_ref, b2_ref, o_ref):
    x = x_ref[...].astype(jnp.bfloat16)
    h = jnp.dot(x, w1_ref[...], preferred_element_type=jnp.float32)
    h = jax.nn.sigmoid(h + b1_ref[...])
    y = jnp.dot(h.astype(jnp.bfloat16), w2_ref[...],
                preferred_element_type=jnp.float32)
    o_ref[...] = (y + b2_ref[...]).astype(o_ref.dtype)


def kernel(x, w1, b1, w2, b2):
    B, D_in = x.shape
    H = w1.shape[1]
    D_out = w2.shape[1]

    tile_b = 16384
    if B <= tile_b:
        tile_b = _round_up(B, 8)
    B_pad = _round_up(B, tile_b)
    if B_pad > B:
        x = jnp.pad(x, ((0, B_pad - B), (0, 0)))

    w1b = w1.astype(jnp.bfloat16)
    w2b = w2.astype(jnp.bfloat16)

    grid = (B_pad // tile_b,)
    out = pl.pallas_call(
        _mlp_kernel,
        out_shape=jax.ShapeDtypeStruct((B_pad, D_out), x.dtype),
        grid_spec=pl.GridSpec(
            grid=grid,
            in_specs=[
                pl.BlockSpec((tile_b, D_in), lambda i: (i, 0)),
                pl.BlockSpec((D_in, H), lambda i: (0, 0)),
                pl.BlockSpec((1, H), lambda i: (0, 0)),
                pl.BlockSpec((H, D_out), lambda i: (0, 0)),
                pl.BlockSpec((1, D_out), lambda i: (0, 0)),
            ],
            out_specs=pl.BlockSpec((tile_b, D_out), lambda i: (i, 0)),
        ),
        compiler_params=pltpu.CompilerParams(
            dimension_semantics=("parallel",),
            vmem_limit_bytes=64 * 1024 * 1024,
        ),
    )(x, w1b, b1, w2b, b2)

    return out[:B] if B_pad > B else out


# speedup vs baseline: 7.6476x; 6.0931x over previous
import jax
import jax.numpy as jnp
from jax.experimental import pallas as pl
from jax.experimental.pallas import tpu as pltpu


def _mlp_t_kernel(xt_ref, w1t_ref, b1t_ref, w2t_ref, b2t_ref, o_ref):
    xt = xt_ref[...].astype(jnp.bfloat16)
    h = jnp.dot(w1t_ref[...], xt, preferred_element_type=jnp.float32)
    h = jax.nn.sigmoid(h + b1t_ref[...])
    y = jnp.dot(w2t_ref[...], h.astype(jnp.bfloat16),
                preferred_element_type=jnp.float32)
    o_ref[...] = (y + b2t_ref[...]).astype(o_ref.dtype)


def kernel(x, w1, b1, w2, b2):
    B, D_in = x.shape
    H = w1.shape[1]
    D_out = w2.shape[1]

    xt = x.T
    w1t = w1.T.astype(jnp.bfloat16)
    w2t = w2.T.astype(jnp.bfloat16)
    b1t = b1.reshape(H, 1)
    b2t = b2.reshape(D_out, 1)

    tile_n = 16384
    if B % tile_n != 0:
        tile_n = 8192 if B % 8192 == 0 else B
    grid = (B // tile_n,)

    out_t = pl.pallas_call(
        _mlp_t_kernel,
        out_shape=jax.ShapeDtypeStruct((D_out, B), x.dtype),
        grid_spec=pl.GridSpec(
            grid=grid,
            in_specs=[
                pl.BlockSpec((D_in, tile_n), lambda i: (0, i)),
                pl.BlockSpec((H, D_in), lambda i: (0, 0)),
                pl.BlockSpec((H, 1), lambda i: (0, 0)),
                pl.BlockSpec((D_out, H), lambda i: (0, 0)),
                pl.BlockSpec((D_out, 1), lambda i: (0, 0)),
            ],
            out_specs=pl.BlockSpec((D_out, tile_n), lambda i: (0, i)),
        ),
        compiler_params=pltpu.CompilerParams(
            dimension_semantics=("parallel",),
            vmem_limit_bytes=64 * 1024 * 1024,
        ),
    )(xt, w1t, b1t, w2t, b2t)

    return out_t.T


# speedup vs baseline: 10.2667x; 1.3425x over previous
import jax
import jax.numpy as jnp
from jax import lax
from jax.experimental import pallas as pl
from jax.experimental.pallas import tpu as pltpu


def _mlp_t_kernel(xt_ref, w1_ref, b1_ref, w2t_ref, b2_ref, o_ref):
    xt = xt_ref[...].astype(jnp.bfloat16)
    w1b = w1_ref[...].astype(jnp.bfloat16)
    h = lax.dot_general(w1b, xt, (((0,), (0,)), ((), ())),
                        preferred_element_type=jnp.float32)
    h = 0.5 * jnp.tanh(0.5 * (h + b1_ref[...].T)) + 0.5
    w2tb = w2t_ref[...].astype(jnp.bfloat16)
    y = jnp.dot(w2tb, h.astype(jnp.bfloat16),
                preferred_element_type=jnp.float32)
    o_ref[...] = (y + b2_ref[...].T).astype(o_ref.dtype)


def kernel(x, w1, b1, w2, b2):
    B, D_in = x.shape
    H = w1.shape[1]
    D_out = w2.shape[1]

    xt = x.T
    w2t = w2.T

    tile_n = 16384
    if B % tile_n != 0:
        tile_n = 8192 if B % 8192 == 0 else B
    grid = (B // tile_n,)

    out_t = pl.pallas_call(
        _mlp_t_kernel,
        out_shape=jax.ShapeDtypeStruct((D_out, B), x.dtype),
        grid_spec=pl.GridSpec(
            grid=grid,
            in_specs=[
                pl.BlockSpec((D_in, tile_n), lambda i: (0, i)),
                pl.BlockSpec((D_in, H), lambda i: (0, 0)),
                pl.BlockSpec((1, H), lambda i: (0, 0)),
                pl.BlockSpec((D_out, H), lambda i: (0, 0)),
                pl.BlockSpec((1, D_out), lambda i: (0, 0)),
            ],
            out_specs=pl.BlockSpec((D_out, tile_n), lambda i: (0, i)),
        ),
        compiler_params=pltpu.CompilerParams(
            dimension_semantics=("parallel",),
            vmem_limit_bytes=64 * 1024 * 1024,
        ),
    )(xt, w1, b1, w2t, b2)

    return out_t.T


# speedup vs baseline: 10.8175x; 1.0536x over previous
import jax
import jax.numpy as jnp
from jax import lax
from jax.experimental import pallas as pl
from jax.experimental.pallas import tpu as pltpu


def _mlp_t_kernel(xt_ref, w1_ref, b1_ref, w2t_ref, b2_ref, o_ref):
    xt = xt_ref[...].astype(jnp.bfloat16)
    w1b = (w1_ref[...] * 0.5).astype(jnp.bfloat16)
    z = lax.dot_general(w1b, xt, (((0,), (0,)), ((), ())),
                        preferred_element_type=jnp.float32)
    t = jnp.tanh(z + (0.5 * b1_ref[...].T))
    w2t = w2t_ref[...]
    w2tb = (w2t * 0.5).astype(jnp.bfloat16)
    c2 = 0.5 * jnp.sum(w2t, axis=1, keepdims=True) + b2_ref[...].T
    y = jnp.dot(w2tb, t.astype(jnp.bfloat16),
                preferred_element_type=jnp.float32)
    o_ref[...] = (y + c2).astype(o_ref.dtype)


def kernel(x, w1, b1, w2, b2):
    B, D_in = x.shape
    H = w1.shape[1]
    D_out = w2.shape[1]

    xt = x.T
    w2t = w2.T

    tile_n = 16384
    if B % tile_n != 0:
        tile_n = 8192 if B % 8192 == 0 else B
    grid = (B // tile_n,)

    out_t = pl.pallas_call(
        _mlp_t_kernel,
        out_shape=jax.ShapeDtypeStruct((D_out, B), x.dtype),
        grid_spec=pl.GridSpec(
            grid=grid,
            in_specs=[
                pl.BlockSpec((D_in, tile_n), lambda i: (0, i)),
                pl.BlockSpec((D_in, H), lambda i: (0, 0)),
                pl.BlockSpec((1, H), lambda i: (0, 0)),
                pl.BlockSpec((D_out, H), lambda i: (0, 0)),
                pl.BlockSpec((1, D_out), lambda i: (0, 0)),
            ],
            out_specs=pl.BlockSpec((D_out, tile_n), lambda i: (0, i)),
        ),
        compiler_params=pltpu.CompilerParams(
            dimension_semantics=("parallel",),
            vmem_limit_bytes=64 * 1024 * 1024,
        ),
    )(xt, w1, b1, w2t, b2)

    return out_t.T


# speedup vs baseline: 12.9933x; 1.2011x over previous
import jax
import jax.numpy as jnp
from jax import lax
from jax.experimental import pallas as pl
from jax.experimental.pallas import tpu as pltpu


def _mlp_t_kernel(xt_ref, w1_ref, b1_ref, w2t_ref, b2_ref, o_ref):
    xt = xt_ref[...].astype(jnp.bfloat16)
    w1b = (w1_ref[...] * 0.5).astype(jnp.bfloat16)
    z = lax.dot_general(w1b, xt, (((0,), (0,)), ((), ())),
                        preferred_element_type=jnp.float32)
    t = jnp.tanh(z + (0.5 * b1_ref[...].T))
    w2t = w2t_ref[...]
    w2tb = (w2t * 0.5).astype(jnp.bfloat16)
    c2 = 0.5 * jnp.sum(w2t, axis=1, keepdims=True) + b2_ref[...].T
    y = jnp.dot(w2tb, t.astype(jnp.bfloat16),
                preferred_element_type=jnp.float32)
    o_ref[...] = (y + c2).astype(o_ref.dtype)


def kernel(x, w1, b1, w2, b2):
    B, D_in = x.shape
    H = w1.shape[1]
    D_out = w2.shape[1]

    xt = x.T
    w2t = w2.T

    tile_n = 32768
    if B % tile_n != 0:
        tile_n = 8192 if B % 8192 == 0 else B
    grid = (B // tile_n,)

    out_t = pl.pallas_call(
        _mlp_t_kernel,
        out_shape=jax.ShapeDtypeStruct((D_out, B), x.dtype),
        grid_spec=pl.GridSpec(
            grid=grid,
            in_specs=[
                pl.BlockSpec((D_in, tile_n), lambda i: (0, i)),
                pl.BlockSpec((D_in, H), lambda i: (0, 0)),
                pl.BlockSpec((1, H), lambda i: (0, 0)),
                pl.BlockSpec((D_out, H), lambda i: (0, 0)),
                pl.BlockSpec((1, D_out), lambda i: (0, 0)),
            ],
            out_specs=pl.BlockSpec((D_out, tile_n), lambda i: (0, i)),
        ),
        compiler_params=pltpu.CompilerParams(
            dimension_semantics=("parallel",),
            vmem_limit_bytes=64 * 1024 * 1024,
        ),
    )(xt, w1, b1, w2t, b2)

    return out_t.T


# speedup vs baseline: 13.8613x; 1.0668x over previous
import jax
import jax.numpy as jnp
from jax import lax
from jax.experimental import pallas as pl
from jax.experimental.pallas import tpu as pltpu


def _mlp_t_kernel(xt_ref, w1_ref, b1_ref, w2t_ref, b2_ref, o_ref):
    xt = xt_ref[...].astype(jnp.bfloat16)
    w1b = (w1_ref[...] * 0.5).astype(jnp.bfloat16)
    z = lax.dot_general(w1b, xt, (((0,), (0,)), ((), ())),
                        preferred_element_type=jnp.float32)
    t = jnp.tanh(z + (0.5 * b1_ref[...].T))
    w2t = w2t_ref[...]
    w2tb = (w2t * 0.5).astype(jnp.bfloat16)
    c2 = 0.5 * jnp.sum(w2t, axis=1, keepdims=True) + b2_ref[...].T
    y = jnp.dot(w2tb, t.astype(jnp.bfloat16),
                preferred_element_type=jnp.float32)
    o_ref[...] = (y + c2).astype(o_ref.dtype)


def kernel(x, w1, b1, w2, b2):
    B, D_in = x.shape
    H = w1.shape[1]
    D_out = w2.shape[1]

    xt = x.T
    w2t = w2.T

    tile_n = 65536
    if B % tile_n != 0:
        tile_n = 8192 if B % 8192 == 0 else B
    grid = (B // tile_n,)

    out_t = pl.pallas_call(
        _mlp_t_kernel,
        out_shape=jax.ShapeDtypeStruct((D_out, B), x.dtype),
        grid_spec=pl.GridSpec(
            grid=grid,
            in_specs=[
                pl.BlockSpec((D_in, tile_n), lambda i: (0, i)),
                pl.BlockSpec((D_in, H), lambda i: (0, 0)),
                pl.BlockSpec((1, H), lambda i: (0, 0)),
                pl.BlockSpec((D_out, H), lambda i: (0, 0)),
                pl.BlockSpec((1, D_out), lambda i: (0, 0)),
            ],
            out_specs=pl.BlockSpec((D_out, tile_n), lambda i: (0, i)),
        ),
        compiler_params=pltpu.CompilerParams(
            dimension_semantics=("parallel",),
            vmem_limit_bytes=64 * 1024 * 1024,
        ),
    )(xt, w1, b1, w2t, b2)

    return out_t.T
